# Initial kernel scaffold; baseline (speedup 1.0000x reference)
import jax
import jax.numpy as jnp
from jax.experimental import pallas as pl


def kernel(high_dim_features, low_dim_features, edge_index, bn_low_g, bn_low_b, bn_high_g, bn_high_b, W_low, b_low, mlp_low_g, mlp_low_b, W_high, b_high, mlp_high_g, mlp_high_b, W_gcn, b_gcn, W_cls, b_cls):
    raise NotImplementedError("write your pallas kernel here")


# speedup vs baseline: 3.9002x; 3.9002x over previous
import functools

import jax
import jax.numpy as jnp
from jax.experimental import pallas as pl
from jax.experimental.pallas import tpu as pltpu

N = 50000
E = 800000
HD = 512
LD = 16
EMB = 32
HID = 64
OUT = 40
EPS = 1e-5

BN_ROWS = 2000
NBLK = N // BN_ROWS


def _stats_body(hi_ref, lo_ref, sh_ref, sl_ref):
    i = pl.program_id(0)

    @pl.when(i == 0)
    def _():
        sh_ref[...] = jnp.zeros_like(sh_ref)
        sl_ref[...] = jnp.zeros_like(sl_ref)

    hi = hi_ref[...]
    lo = lo_ref[...]
    sh_ref[0, :] += jnp.sum(hi, axis=0)
    sh_ref[1, :] += jnp.sum(hi * hi, axis=0)
    sl_ref[0, :] += jnp.sum(lo, axis=0)
    sl_ref[1, :] += jnp.sum(lo * lo, axis=0)


def _embed_body(hi_ref, lo_ref, wh_ref, bh_ref, wl_ref, bl_ref, y_ref, sy_ref):
    i = pl.program_id(0)

    @pl.when(i == 0)
    def _():
        sy_ref[...] = jnp.zeros_like(sy_ref)

    yh = jnp.maximum(
        jnp.dot(hi_ref[...], wh_ref[...], preferred_element_type=jnp.float32)
        + bh_ref[0, :], 0.0)
    yl = jnp.maximum(
        jnp.dot(lo_ref[...], wl_ref[...], preferred_element_type=jnp.float32)
        + bl_ref[0, :], 0.0)
    y = jnp.concatenate([yh, yl], axis=1)
    y_ref[...] = y
    sy_ref[0, :] += jnp.sum(y, axis=0)
    sy_ref[1, :] += jnp.sum(y * y, axis=0)


def _project_body(y_ref, w2_ref, c2_ref, deg_ref, g_ref):
    h = jnp.dot(y_ref[...], w2_ref[...], preferred_element_type=jnp.float32)
    h = h + c2_ref[0, :]
    dinv = jax.lax.rsqrt(deg_ref[...])
    g_ref[...] = h * dinv


def _final_body(a0_ref, a1_ref, g_ref, deg_ref, bg_ref, wc_ref, bc_ref, o_ref):
    acc = jnp.concatenate([a0_ref[0], a1_ref[0]], axis=1)
    dinv = jax.lax.rsqrt(deg_ref[...])
    z = jnp.tanh(dinv * (acc + g_ref[...]) + bg_ref[0, :])
    logits = jnp.dot(z, wc_ref[...], preferred_element_type=jnp.float32)
    logits = logits + bc_ref[0, :]
    m = jnp.max(logits, axis=1, keepdims=True)
    lse = m + jnp.log(jnp.sum(jnp.exp(logits - m), axis=1, keepdims=True))
    o_ref[...] = logits - lse


def _fold(gamma, beta, s1, s2):
    m = s1 / N
    v = s2 / N - m * m
    s = gamma * jax.lax.rsqrt(v + EPS)
    return s, beta - m * s


def kernel(high_dim_features, low_dim_features, edge_index,
           bn_low_g, bn_low_b, bn_high_g, bn_high_b,
           W_low, b_low, mlp_low_g, mlp_low_b,
           W_high, b_high, mlp_high_g, mlp_high_b,
           W_gcn, b_gcn, W_cls, b_cls):
    row = edge_index[0]
    col = edge_index[1]

    sh, sl = pl.pallas_call(
        _stats_body,
        grid=(NBLK,),
        in_specs=[
            pl.BlockSpec((BN_ROWS, HD), lambda i: (i, 0)),
            pl.BlockSpec((BN_ROWS, LD), lambda i: (i, 0)),
        ],
        out_specs=[
            pl.BlockSpec((8, HD), lambda i: (0, 0)),
            pl.BlockSpec((8, LD), lambda i: (0, 0)),
        ],
        out_shape=[
            jax.ShapeDtypeStruct((8, HD), jnp.float32),
            jax.ShapeDtypeStruct((8, LD), jnp.float32),
        ],
    )(high_dim_features, low_dim_features)

    s_hi, t_hi = _fold(bn_high_g, bn_high_b, sh[0], sh[1])
    s_lo, t_lo = _fold(bn_low_g, bn_low_b, sl[0], sl[1])
    Wh = s_hi[:, None] * W_high
    bh = (t_hi @ W_high + b_high)[None, :]
    Wl = s_lo[:, None] * W_low
    bl = (t_lo @ W_low + b_low)[None, :]

    y, sy = pl.pallas_call(
        _embed_body,
        grid=(NBLK,),
        in_specs=[
            pl.BlockSpec((BN_ROWS, HD), lambda i: (i, 0)),
            pl.BlockSpec((BN_ROWS, LD), lambda i: (i, 0)),
            pl.BlockSpec((HD, EMB), lambda i: (0, 0)),
            pl.BlockSpec((1, EMB), lambda i: (0, 0)),
            pl.BlockSpec((LD, EMB), lambda i: (0, 0)),
            pl.BlockSpec((1, EMB), lambda i: (0, 0)),
        ],
        out_specs=[
            pl.BlockSpec((BN_ROWS, HID), lambda i: (i, 0)),
            pl.BlockSpec((8, HID), lambda i: (0, 0)),
        ],
        out_shape=[
            jax.ShapeDtypeStruct((N, HID), jnp.float32),
            jax.ShapeDtypeStruct((8, HID), jnp.float32),
        ],
    )(high_dim_features, low_dim_features, Wh, bh, Wl, bl)

    gy = jnp.concatenate([mlp_high_g, mlp_low_g])
    by = jnp.concatenate([mlp_high_b, mlp_low_b])
    s_y, t_y = _fold(gy, by, sy[0], sy[1])
    W2 = s_y[:, None] * W_gcn
    c2 = (t_y @ W_gcn)[None, :]

    deg = jnp.ones((N,), jnp.float32).at[col].add(1.0)
    deg2d = deg[:, None]

    g = pl.pallas_call(
        _project_body,
        grid=(NBLK,),
        in_specs=[
            pl.BlockSpec((BN_ROWS, HID), lambda i: (i, 0)),
            pl.BlockSpec((HID, HID), lambda i: (0, 0)),
            pl.BlockSpec((1, HID), lambda i: (0, 0)),
            pl.BlockSpec((BN_ROWS, 1), lambda i: (i, 0)),
        ],
        out_specs=pl.BlockSpec((BN_ROWS, HID), lambda i: (i, 0)),
        out_shape=jax.ShapeDtypeStruct((N, HID), jnp.float32),
    )(y, W2, c2, deg2d)

    acc = jnp.zeros((N, HID), jnp.float32).at[col].add(g[row])
    acc = acc.reshape(N, 2, EMB).transpose(1, 0, 2)

    out = pl.pallas_call(
        _final_body,
        grid=(NBLK,),
        in_specs=[
            pl.BlockSpec((1, BN_ROWS, EMB), lambda i: (0, i, 0)),
            pl.BlockSpec((1, BN_ROWS, EMB), lambda i: (1, i, 0)),
            pl.BlockSpec((BN_ROWS, HID), lambda i: (i, 0)),
            pl.BlockSpec((BN_ROWS, 1), lambda i: (i, 0)),
            pl.BlockSpec((1, HID), lambda i: (0, 0)),
            pl.BlockSpec((HID, OUT), lambda i: (0, 0)),
            pl.BlockSpec((1, OUT), lambda i: (0, 0)),
        ],
        out_specs=pl.BlockSpec((BN_ROWS, OUT), lambda i: (i, 0)),
        out_shape=jax.ShapeDtypeStruct((N, OUT), jnp.float32),
    )(acc, acc, g, deg2d, b_gcn[None, :], W_cls, b_cls[None, :])

    return out


# speedup vs baseline: 27.1933x; 6.9723x over previous
import jax
import jax.numpy as jnp
from jax import lax
from jax.experimental import pallas as pl
from jax.experimental.pallas import tpu as pltpu
from jax.experimental.pallas import tpu_sc as plsc

N = 50000
E = 800000
HD = 512
LD = 16
EMB = 32
HID = 64
OUT = 40
EPS = 1e-5

BN_ROWS = 2000
NBLK = N // BN_ROWS

NSC = 2
NSUB = 16
DEGW = 16
NP = 50176
DSTRIPE = NP // NSUB
DCH = 1000
DSPAN = E // (NSC * NSUB)

MCH = 400
MSPAN = E // NSUB
MSTRIPE = N // NSUB


def _stats_body(hi_ref, lo_ref, sh_ref, sl_ref):
    i = pl.program_id(0)

    @pl.when(i == 0)
    def _():
        sh_ref[...] = jnp.zeros_like(sh_ref)
        sl_ref[...] = jnp.zeros_like(sl_ref)

    hi = hi_ref[...]
    lo = lo_ref[...]
    sh_ref[0, :] += jnp.sum(hi, axis=0)
    sh_ref[1, :] += jnp.sum(hi * hi, axis=0)
    sl_ref[0, :] += jnp.sum(lo, axis=0)
    sl_ref[1, :] += jnp.sum(lo * lo, axis=0)


def _embed_body(hi_ref, lo_ref, wh_ref, bh_ref, wl_ref, bl_ref, y_ref, sy_ref):
    i = pl.program_id(0)

    @pl.when(i == 0)
    def _():
        sy_ref[...] = jnp.zeros_like(sy_ref)

    yh = jnp.maximum(
        jnp.dot(hi_ref[...], wh_ref[...], preferred_element_type=jnp.float32)
        + bh_ref[0, :], 0.0)
    yl = jnp.maximum(
        jnp.dot(lo_ref[...], wl_ref[...], preferred_element_type=jnp.float32)
        + bl_ref[0, :], 0.0)
    y = jnp.concatenate([yh, yl], axis=1)
    y_ref[...] = y
    sy_ref[0, :] += jnp.sum(y, axis=0)
    sy_ref[1, :] += jnp.sum(y * y, axis=0)


def _project_body(y_ref, w2_ref, c2_ref, p0_ref, p1_ref, g0_ref, g1_ref):
    h = jnp.dot(y_ref[...], w2_ref[...], preferred_element_type=jnp.float32)
    h = h + c2_ref[0, :]
    deg = 1.0 + p0_ref[0, :, :1] + p1_ref[0, :, :1]
    g = h * lax.rsqrt(deg)
    g0_ref[...] = g[:, :EMB]
    g1_ref[...] = g[:, EMB:]


def _final_body(a0_ref, a1_ref, p0_ref, p1_ref, bg_ref, wc_ref, bc_ref, o_ref):
    acc = jnp.concatenate([a0_ref[...], a1_ref[...]], axis=1)
    deg = 1.0 + p0_ref[0, :, :1] + p1_ref[0, :, :1]
    z = jnp.tanh(lax.rsqrt(deg) * acc + bg_ref[0, :])
    logits = jnp.dot(z, wc_ref[...], preferred_element_type=jnp.float32)
    logits = logits + bc_ref[0, :]
    m = jnp.max(logits, axis=1, keepdims=True)
    lse = m + jnp.log(jnp.sum(jnp.exp(logits - m), axis=1, keepdims=True))
    o_ref[...] = logits - lse


def _sc_mesh():
    return plsc.VectorSubcoreMesh(core_axis_name="c", subcore_axis_name="s")


_SC_PARAMS = pltpu.CompilerParams(use_tc_tiling_on_sc=False)


def _degree_sc(edge_index):
    ei_flat = edge_index.reshape(2 * E)

    @pl.kernel(
        out_type=jax.ShapeDtypeStruct((NSC, NP, DEGW), jnp.float32),
        mesh=_sc_mesh(),
        scratch_types=[
            pltpu.VMEM_SHARED((NP, DEGW), jnp.float32),
            pltpu.VMEM((DCH,), jnp.int32),
            pltpu.VMEM((DCH, DEGW), jnp.float32),
            pltpu.VMEM((DSTRIPE, DEGW), jnp.float32),
        ],
        compiler_params=_SC_PARAMS,
    )
    def deg_kernel(ei_hbm, deg_hbm, deg_s, cidx, ones_t, zeros_t):
        c = lax.axis_index("c")
        s = lax.axis_index("s")

        @pl.loop(0, DCH)
        def _(i):
            ones_t[i, :] = jnp.ones((DEGW,), jnp.float32)

        @pl.loop(0, DSTRIPE)
        def _(i):
            zeros_t[i, :] = jnp.zeros((DEGW,), jnp.float32)

        pltpu.sync_copy(zeros_t, deg_s.at[pl.ds(s * DSTRIPE, DSTRIPE)])
        plsc.subcore_barrier()

        w = c * NSUB + s

        @pl.loop(0, DSPAN // DCH)
        def _(j):
            base = w * DSPAN + j * DCH
            pltpu.sync_copy(ei_hbm.at[pl.ds(E + base, DCH)], cidx)
            pltpu.sync_copy(ones_t, deg_s.at[cidx], add=True)

        plsc.subcore_barrier()
        pltpu.sync_copy(deg_s.at[pl.ds(s * DSTRIPE, DSTRIPE)],
                        deg_hbm.at[c, pl.ds(s * DSTRIPE, DSTRIPE)])

    return deg_kernel(ei_flat)


def _message_sc(g0, g1, edge_index):
    @pl.kernel(
        out_type=[jax.ShapeDtypeStruct((N, EMB), jnp.float32),
                  jax.ShapeDtypeStruct((N, EMB), jnp.float32)],
        mesh=_sc_mesh(),
        scratch_types=[
            pltpu.VMEM_SHARED((N, EMB), jnp.float32),
            pltpu.VMEM((MCH,), jnp.int32),
            pltpu.VMEM((MCH,), jnp.int32),
            pltpu.VMEM((MCH, EMB), jnp.float32),
            pltpu.SemaphoreType.DMA,
        ],
        compiler_params=_SC_PARAMS,
    )
    def msg_kernel(g0_hbm, g1_hbm, ei_hbm, a0_hbm, a1_hbm,
                   acc_s, ridx, cidx, msg, sem):
        c = lax.axis_index("c")
        s = lax.axis_index("s")

        def run(g_hbm, a_hbm):
            stripe = pl.ds(s * MSTRIPE, MSTRIPE)
            pltpu.sync_copy(g_hbm.at[stripe], acc_s.at[stripe])
            plsc.subcore_barrier()

            @pl.loop(0, MSPAN // MCH)
            def _(j):
                base = s * MSPAN + j * MCH
                pltpu.sync_copy(ei_hbm.at[pl.ds(base, MCH)], ridx)
                pltpu.sync_copy(ei_hbm.at[pl.ds(E + base, MCH)], cidx)
                pltpu.async_copy(g_hbm.at[ridx], msg, sem).wait()
                pltpu.sync_copy(msg, acc_s.at[cidx], add=True)

            plsc.subcore_barrier()
            pltpu.sync_copy(acc_s.at[stripe], a_hbm.at[stripe])

        @pl.when(c == 0)
        def _():
            run(g0_hbm, a0_hbm)

        @pl.when(c == 1)
        def _():
            run(g1_hbm, a1_hbm)

    return msg_kernel(g0, g1, edge_index.reshape(2 * E))


def _fold(gamma, beta, s1, s2):
    m = s1 / N
    v = s2 / N - m * m
    s = gamma * lax.rsqrt(v + EPS)
    return s, beta - m * s


def kernel(high_dim_features, low_dim_features, edge_index,
           bn_low_g, bn_low_b, bn_high_g, bn_high_b,
           W_low, b_low, mlp_low_g, mlp_low_b,
           W_high, b_high, mlp_high_g, mlp_high_b,
           W_gcn, b_gcn, W_cls, b_cls):
    degp = _degree_sc(edge_index)

    sh, sl = pl.pallas_call(
        _stats_body,
        grid=(NBLK,),
        in_specs=[
            pl.BlockSpec((BN_ROWS, HD), lambda i: (i, 0)),
            pl.BlockSpec((BN_ROWS, LD), lambda i: (i, 0)),
        ],
        out_specs=[
            pl.BlockSpec((8, HD), lambda i: (0, 0)),
            pl.BlockSpec((8, LD), lambda i: (0, 0)),
        ],
        out_shape=[
            jax.ShapeDtypeStruct((8, HD), jnp.float32),
            jax.ShapeDtypeStruct((8, LD), jnp.float32),
        ],
    )(high_dim_features, low_dim_features)

    s_hi, t_hi = _fold(bn_high_g, bn_high_b, sh[0], sh[1])
    s_lo, t_lo = _fold(bn_low_g, bn_low_b, sl[0], sl[1])
    Wh = s_hi[:, None] * W_high
    bh = (t_hi @ W_high + b_high)[None, :]
    Wl = s_lo[:, None] * W_low
    bl = (t_lo @ W_low + b_low)[None, :]

    y, sy = pl.pallas_call(
        _embed_body,
        grid=(NBLK,),
        in_specs=[
            pl.BlockSpec((BN_ROWS, HD), lambda i: (i, 0)),
            pl.BlockSpec((BN_ROWS, LD), lambda i: (i, 0)),
            pl.BlockSpec((HD, EMB), lambda i: (0, 0)),
            pl.BlockSpec((1, EMB), lambda i: (0, 0)),
            pl.BlockSpec((LD, EMB), lambda i: (0, 0)),
            pl.BlockSpec((1, EMB), lambda i: (0, 0)),
        ],
        out_specs=[
            pl.BlockSpec((BN_ROWS, HID), lambda i: (i, 0)),
            pl.BlockSpec((8, HID), lambda i: (0, 0)),
        ],
        out_shape=[
            jax.ShapeDtypeStruct((N, HID), jnp.float32),
            jax.ShapeDtypeStruct((8, HID), jnp.float32),
        ],
    )(high_dim_features, low_dim_features, Wh, bh, Wl, bl)

    gy = jnp.concatenate([mlp_high_g, mlp_low_g])
    by = jnp.concatenate([mlp_high_b, mlp_low_b])
    s_y, t_y = _fold(gy, by, sy[0], sy[1])
    W2 = s_y[:, None] * W_gcn
    c2 = (t_y @ W_gcn)[None, :]

    g0, g1 = pl.pallas_call(
        _project_body,
        grid=(NBLK,),
        in_specs=[
            pl.BlockSpec((BN_ROWS, HID), lambda i: (i, 0)),
            pl.BlockSpec((HID, HID), lambda i: (0, 0)),
            pl.BlockSpec((1, HID), lambda i: (0, 0)),
            pl.BlockSpec((1, BN_ROWS, DEGW), lambda i: (0, i, 0)),
            pl.BlockSpec((1, BN_ROWS, DEGW), lambda i: (1, i, 0)),
        ],
        out_specs=[
            pl.BlockSpec((BN_ROWS, EMB), lambda i: (i, 0)),
            pl.BlockSpec((BN_ROWS, EMB), lambda i: (i, 0)),
        ],
        out_shape=[
            jax.ShapeDtypeStruct((N, EMB), jnp.float32),
            jax.ShapeDtypeStruct((N, EMB), jnp.float32),
        ],
    )(y, W2, c2, degp, degp)

    acc0, acc1 = _message_sc(g0, g1, edge_index)

    out = pl.pallas_call(
        _final_body,
        grid=(NBLK,),
        in_specs=[
            pl.BlockSpec((BN_ROWS, EMB), lambda i: (i, 0)),
            pl.BlockSpec((BN_ROWS, EMB), lambda i: (i, 0)),
            pl.BlockSpec((1, BN_ROWS, DEGW), lambda i: (0, i, 0)),
            pl.BlockSpec((1, BN_ROWS, DEGW), lambda i: (1, i, 0)),
            pl.BlockSpec((1, HID), lambda i: (0, 0)),
            pl.BlockSpec((HID, OUT), lambda i: (0, 0)),
            pl.BlockSpec((1, OUT), lambda i: (0, 0)),
        ],
        out_specs=pl.BlockSpec((BN_ROWS, OUT), lambda i: (i, 0)),
        out_shape=jax.ShapeDtypeStruct((N, OUT), jnp.float32),
    )(acc0, acc1, degp, degp, b_gcn[None, :], W_cls, b_cls[None, :])

    return out


# speedup vs baseline: 33.3903x; 1.2279x over previous
import jax
import jax.numpy as jnp
from jax import lax
from jax.experimental import pallas as pl
from jax.experimental.pallas import tpu as pltpu
from jax.experimental.pallas import tpu_sc as plsc

N = 50000
E = 800000
HD = 512
LD = 16
EMB = 32
HID = 64
OUT = 40
EPS = 1e-5

BN_ROWS = 2000
NBLK = N // BN_ROWS

NSC = 2
NSUB = 16
DEGW = 16
NP = 50176
DSTRIPE = NP // NSUB
DCH = 1000
DSPAN = E // (NSC * NSUB)

MCH = 400
MSPAN = E // NSUB
MSTRIPE = N // NSUB


def _stats_body(hi_ref, lo_ref, sh_ref, sl_ref):
    i = pl.program_id(0)

    @pl.when(i == 0)
    def _():
        sh_ref[...] = jnp.zeros_like(sh_ref)
        sl_ref[...] = jnp.zeros_like(sl_ref)

    hi = hi_ref[...]
    lo = lo_ref[...]
    sh_ref[0, :] += jnp.sum(hi, axis=0)
    sh_ref[1, :] += jnp.sum(hi * hi, axis=0)
    sl_ref[0, :] += jnp.sum(lo, axis=0)
    sl_ref[1, :] += jnp.sum(lo * lo, axis=0)


def _embed_body(hi_ref, lo_ref, wh_ref, bh_ref, wl_ref, bl_ref, y_ref, sy_ref):
    i = pl.program_id(0)

    @pl.when(i == 0)
    def _():
        sy_ref[...] = jnp.zeros_like(sy_ref)

    yh = jnp.maximum(
        jnp.dot(hi_ref[...], wh_ref[...], preferred_element_type=jnp.float32)
        + bh_ref[0, :], 0.0)
    yl = jnp.maximum(
        jnp.dot(lo_ref[...], wl_ref[...], preferred_element_type=jnp.float32)
        + bl_ref[0, :], 0.0)
    y = jnp.concatenate([yh, yl], axis=1)
    y_ref[...] = y
    sy_ref[0, :] += jnp.sum(y, axis=0)
    sy_ref[1, :] += jnp.sum(y * y, axis=0)


def _project_body(y_ref, w2_ref, c2_ref, p0_ref, p1_ref, g0_ref, g1_ref):
    h = jnp.dot(y_ref[...], w2_ref[...], preferred_element_type=jnp.float32)
    h = h + c2_ref[0, :]
    deg = 1.0 + p0_ref[0, :, :1] + p1_ref[0, :, :1]
    g = h * lax.rsqrt(deg)
    g0_ref[...] = g[:, :EMB]
    g1_ref[...] = g[:, EMB:]


def _final_body(a0_ref, a1_ref, p0_ref, p1_ref, bg_ref, wc_ref, bc_ref, o_ref):
    acc = jnp.concatenate([a0_ref[...], a1_ref[...]], axis=1)
    deg = 1.0 + p0_ref[0, :, :1] + p1_ref[0, :, :1]
    z = jnp.tanh(lax.rsqrt(deg) * acc + bg_ref[0, :])
    logits = jnp.dot(z, wc_ref[...], preferred_element_type=jnp.float32)
    logits = logits + bc_ref[0, :]
    m = jnp.max(logits, axis=1, keepdims=True)
    lse = m + jnp.log(jnp.sum(jnp.exp(logits - m), axis=1, keepdims=True))
    o_ref[...] = logits - lse


def _sc_mesh():
    return plsc.VectorSubcoreMesh(core_axis_name="c", subcore_axis_name="s")


_SC_PARAMS = pltpu.CompilerParams(use_tc_tiling_on_sc=False)


def _degree_sc(edge_index):
    ei_flat = edge_index.reshape(2 * E)

    @pl.kernel(
        out_type=jax.ShapeDtypeStruct((NSC, NP, DEGW), jnp.float32),
        mesh=_sc_mesh(),
        scratch_types=[
            pltpu.VMEM_SHARED((NP, DEGW), jnp.float32),
            pltpu.VMEM((DCH,), jnp.int32),
            pltpu.VMEM((DCH, DEGW), jnp.float32),
            pltpu.VMEM((DSTRIPE, DEGW), jnp.float32),
        ],
        compiler_params=_SC_PARAMS,
    )
    def deg_kernel(ei_hbm, deg_hbm, deg_s, cidx, ones_t, zeros_t):
        c = lax.axis_index("c")
        s = lax.axis_index("s")

        @pl.loop(0, DCH)
        def _(i):
            ones_t[i, :] = jnp.ones((DEGW,), jnp.float32)

        @pl.loop(0, DSTRIPE)
        def _(i):
            zeros_t[i, :] = jnp.zeros((DEGW,), jnp.float32)

        pltpu.sync_copy(zeros_t, deg_s.at[pl.ds(s * DSTRIPE, DSTRIPE)])
        plsc.subcore_barrier()

        w = c * NSUB + s

        @pl.loop(0, DSPAN // DCH)
        def _(j):
            base = w * DSPAN + j * DCH
            pltpu.sync_copy(ei_hbm.at[pl.ds(E + base, DCH)], cidx)
            pltpu.sync_copy(ones_t, deg_s.at[cidx], add=True)

        plsc.subcore_barrier()
        pltpu.sync_copy(deg_s.at[pl.ds(s * DSTRIPE, DSTRIPE)],
                        deg_hbm.at[c, pl.ds(s * DSTRIPE, DSTRIPE)])

    return deg_kernel(ei_flat)


def _message_sc(g0, g1, edge_index):
    @pl.kernel(
        out_type=[jax.ShapeDtypeStruct((N, EMB), jnp.float32),
                  jax.ShapeDtypeStruct((N, EMB), jnp.float32)],
        mesh=_sc_mesh(),
        scratch_types=[
            pltpu.VMEM_SHARED((N, EMB), jnp.float32),
            pltpu.VMEM((MCH,), jnp.int32),
            pltpu.VMEM((MCH,), jnp.int32),
            pltpu.VMEM((MCH,), jnp.int32),
            pltpu.VMEM((MCH,), jnp.int32),
            pltpu.VMEM((MCH, EMB), jnp.float32),
            pltpu.VMEM((MCH, EMB), jnp.float32),
            pltpu.SemaphoreType.DMA,
            pltpu.SemaphoreType.DMA,
        ],
        compiler_params=_SC_PARAMS,
    )
    def msg_kernel(g0_hbm, g1_hbm, ei_hbm, a0_hbm, a1_hbm,
                   acc_s, ridx0, cidx0, ridx1, cidx1, msg0, msg1, sem0, sem1):
        c = lax.axis_index("c")
        s = lax.axis_index("s")

        def run(g_hbm, a_hbm):
            stripe = pl.ds(s * MSTRIPE, MSTRIPE)
            pltpu.sync_copy(g_hbm.at[stripe], acc_s.at[stripe])
            plsc.subcore_barrier()

            bufs = ((ridx0, cidx0, msg0, sem0), (ridx1, cidx1, msg1, sem1))

            def start(j, b):
                ridx, cidx, msg, sem = bufs[b]
                base = s * MSPAN + j * MCH
                pltpu.sync_copy(ei_hbm.at[pl.ds(base, MCH)], ridx)
                pltpu.sync_copy(ei_hbm.at[pl.ds(E + base, MCH)], cidx)
                pltpu.async_copy(g_hbm.at[ridx], msg, sem)

            def drain(b):
                ridx, cidx, msg, sem = bufs[b]
                pltpu.make_async_copy(g_hbm.at[ridx], msg, sem).wait()
                pltpu.sync_copy(msg, acc_s.at[cidx], add=True)

            start(0, 0)

            @pl.loop(0, (MSPAN // MCH - 1) // 2)
            def _(i):
                start(2 * i + 1, 1)
                drain(0)
                start(2 * i + 2, 0)
                drain(1)

            drain(0)

            plsc.subcore_barrier()
            pltpu.sync_copy(acc_s.at[stripe], a_hbm.at[stripe])

        @pl.when(c == 0)
        def _():
            run(g0_hbm, a0_hbm)

        @pl.when(c == 1)
        def _():
            run(g1_hbm, a1_hbm)

    return msg_kernel(g0, g1, edge_index.reshape(2 * E))


def _fold(gamma, beta, s1, s2):
    m = s1 / N
    v = s2 / N - m * m
    s = gamma * lax.rsqrt(v + EPS)
    return s, beta - m * s


def kernel(high_dim_features, low_dim_features, edge_index,
           bn_low_g, bn_low_b, bn_high_g, bn_high_b,
           W_low, b_low, mlp_low_g, mlp_low_b,
           W_high, b_high, mlp_high_g, mlp_high_b,
           W_gcn, b_gcn, W_cls, b_cls):
    degp = _degree_sc(edge_index)

    sh, sl = pl.pallas_call(
        _stats_body,
        grid=(NBLK,),
        in_specs=[
            pl.BlockSpec((BN_ROWS, HD), lambda i: (i, 0)),
            pl.BlockSpec((BN_ROWS, LD), lambda i: (i, 0)),
        ],
        out_specs=[
            pl.BlockSpec((8, HD), lambda i: (0, 0)),
            pl.BlockSpec((8, LD), lambda i: (0, 0)),
        ],
        out_shape=[
            jax.ShapeDtypeStruct((8, HD), jnp.float32),
            jax.ShapeDtypeStruct((8, LD), jnp.float32),
        ],
    )(high_dim_features, low_dim_features)

    s_hi, t_hi = _fold(bn_high_g, bn_high_b, sh[0], sh[1])
    s_lo, t_lo = _fold(bn_low_g, bn_low_b, sl[0], sl[1])
    Wh = s_hi[:, None] * W_high
    bh = (t_hi @ W_high + b_high)[None, :]
    Wl = s_lo[:, None] * W_low
    bl = (t_lo @ W_low + b_low)[None, :]

    y, sy = pl.pallas_call(
        _embed_body,
        grid=(NBLK,),
        in_specs=[
            pl.BlockSpec((BN_ROWS, HD), lambda i: (i, 0)),
            pl.BlockSpec((BN_ROWS, LD), lambda i: (i, 0)),
            pl.BlockSpec((HD, EMB), lambda i: (0, 0)),
            pl.BlockSpec((1, EMB), lambda i: (0, 0)),
            pl.BlockSpec((LD, EMB), lambda i: (0, 0)),
            pl.BlockSpec((1, EMB), lambda i: (0, 0)),
        ],
        out_specs=[
            pl.BlockSpec((BN_ROWS, HID), lambda i: (i, 0)),
            pl.BlockSpec((8, HID), lambda i: (0, 0)),
        ],
        out_shape=[
            jax.ShapeDtypeStruct((N, HID), jnp.float32),
            jax.ShapeDtypeStruct((8, HID), jnp.float32),
        ],
    )(high_dim_features, low_dim_features, Wh, bh, Wl, bl)

    gy = jnp.concatenate([mlp_high_g, mlp_low_g])
    by = jnp.concatenate([mlp_high_b, mlp_low_b])
    s_y, t_y = _fold(gy, by, sy[0], sy[1])
    W2 = s_y[:, None] * W_gcn
    c2 = (t_y @ W_gcn)[None, :]

    g0, g1 = pl.pallas_call(
        _project_body,
        grid=(NBLK,),
        in_specs=[
            pl.BlockSpec((BN_ROWS, HID), lambda i: (i, 0)),
            pl.BlockSpec((HID, HID), lambda i: (0, 0)),
            pl.BlockSpec((1, HID), lambda i: (0, 0)),
            pl.BlockSpec((1, BN_ROWS, DEGW), lambda i: (0, i, 0)),
            pl.BlockSpec((1, BN_ROWS, DEGW), lambda i: (1, i, 0)),
        ],
        out_specs=[
            pl.BlockSpec((BN_ROWS, EMB), lambda i: (i, 0)),
            pl.BlockSpec((BN_ROWS, EMB), lambda i: (i, 0)),
        ],
        out_shape=[
            jax.ShapeDtypeStruct((N, EMB), jnp.float32),
            jax.ShapeDtypeStruct((N, EMB), jnp.float32),
        ],
    )(y, W2, c2, degp, degp)

    acc0, acc1 = _message_sc(g0, g1, edge_index)

    out = pl.pallas_call(
        _final_body,
        grid=(NBLK,),
        in_specs=[
            pl.BlockSpec((BN_ROWS, EMB), lambda i: (i, 0)),
            pl.BlockSpec((BN_ROWS, EMB), lambda i: (i, 0)),
            pl.BlockSpec((1, BN_ROWS, DEGW), lambda i: (0, i, 0)),
            pl.BlockSpec((1, BN_ROWS, DEGW), lambda i: (1, i, 0)),
            pl.BlockSpec((1, HID), lambda i: (0, 0)),
            pl.BlockSpec((HID, OUT), lambda i: (0, 0)),
            pl.BlockSpec((1, OUT), lambda i: (0, 0)),
        ],
        out_specs=pl.BlockSpec((BN_ROWS, OUT), lambda i: (i, 0)),
        out_shape=jax.ShapeDtypeStruct((N, OUT), jnp.float32),
    )(acc0, acc1, degp, degp, b_gcn[None, :], W_cls, b_cls[None, :])

    return out


# speedup vs baseline: 36.7861x; 1.1017x over previous
import jax
import jax.numpy as jnp
from jax import lax
from jax.experimental import pallas as pl
from jax.experimental.pallas import tpu as pltpu
from jax.experimental.pallas import tpu_sc as plsc

N = 50000
E = 800000
HD = 512
LD = 16
EMB = 32
HID = 64
OUT = 40
EPS = 1e-5

BN_ROWS = 2000
NBLK = N // BN_ROWS

NSC = 2
NSUB = 16
DEGW = 16
NP = 50176
DSTRIPE = NP // NSUB

MCH = 400
NCHT = E // MCH
MSPANC = NCHT // NSUB
MSTRIPE = N // NSUB
DCHUNKS = NCHT // (NSC * NSUB)
DREM = NCHT - DCHUNKS * NSC * NSUB


def _stats_body(hi_ref, lo_ref, sh_ref, sl_ref):
    i = pl.program_id(0)

    @pl.when(i == 0)
    def _():
        sh_ref[...] = jnp.zeros_like(sh_ref)
        sl_ref[...] = jnp.zeros_like(sl_ref)

    hi = hi_ref[...]
    lo = lo_ref[...]
    sh_ref[0, :] += jnp.sum(hi, axis=0)
    sh_ref[1, :] += jnp.sum(hi * hi, axis=0)
    sl_ref[0, :] += jnp.sum(lo, axis=0)
    sl_ref[1, :] += jnp.sum(lo * lo, axis=0)


def _embed_body(hi_ref, lo_ref, wh_ref, bh_ref, wl_ref, bl_ref, y_ref, sy_ref):
    i = pl.program_id(0)

    @pl.when(i == 0)
    def _():
        sy_ref[...] = jnp.zeros_like(sy_ref)

    yh = jnp.maximum(
        jnp.dot(hi_ref[...], wh_ref[...], preferred_element_type=jnp.float32)
        + bh_ref[0, :], 0.0)
    yl = jnp.maximum(
        jnp.dot(lo_ref[...], wl_ref[...], preferred_element_type=jnp.float32)
        + bl_ref[0, :], 0.0)
    y = jnp.concatenate([yh, yl], axis=1)
    y_ref[...] = y
    sy_ref[0, :] += jnp.sum(y, axis=0)
    sy_ref[1, :] += jnp.sum(y * y, axis=0)


def _dinv_col(deg_ref):
    d = jnp.reshape(deg_ref[0, 0, :], (BN_ROWS, 1))
    return lax.rsqrt(1.0 + d)


def _project_body(y_ref, w2_ref, c2_ref, deg_ref, g0_ref, g1_ref):
    h = jnp.dot(y_ref[...], w2_ref[...], preferred_element_type=jnp.float32)
    h = h + c2_ref[0, :]
    g = h * _dinv_col(deg_ref)
    g0_ref[...] = g[:, :EMB]
    g1_ref[...] = g[:, EMB:]


def _final_body(a0_ref, a1_ref, deg_ref, bg_ref, wc_ref, bc_ref, o_ref):
    acc = jnp.concatenate([a0_ref[...], a1_ref[...]], axis=1)
    z = jnp.tanh(_dinv_col(deg_ref) * acc + bg_ref[0, :])
    logits = jnp.dot(z, wc_ref[...], preferred_element_type=jnp.float32)
    logits = logits + bc_ref[0, :]
    m = jnp.max(logits, axis=1, keepdims=True)
    lse = m + jnp.log(jnp.sum(jnp.exp(logits - m), axis=1, keepdims=True))
    o_ref[...] = logits - lse


def _sc_mesh():
    return plsc.VectorSubcoreMesh(core_axis_name="c", subcore_axis_name="s")


_SC_PARAMS = pltpu.CompilerParams(use_tc_tiling_on_sc=False)
_SC_PARAMS_NL = pltpu.CompilerParams(use_tc_tiling_on_sc=False,
                                     needs_layout_passes=False)


def _degree_sc(ei_pack):
    @pl.kernel(
        out_type=jax.ShapeDtypeStruct((NSC, NP), jnp.float32),
        mesh=_sc_mesh(),
        scratch_types=[
            pltpu.VMEM_SHARED((NP, DEGW), jnp.float32),
            pltpu.VMEM((MCH,), jnp.int32),
            pltpu.VMEM((MCH, DEGW), jnp.float32),
            pltpu.VMEM((DSTRIPE, DEGW), jnp.float32),
            pltpu.VMEM((DSTRIPE,), jnp.float32),
        ],
        compiler_params=_SC_PARAMS_NL,
    )
    def deg_kernel(ei_hbm, deg_hbm, deg_s, cidx, ones_t, stripe_t, out1d):
        c = lax.axis_index("c")
        s = lax.axis_index("s")

        @pl.loop(0, MCH)
        def _(i):
            ones_t[i, :] = jnp.ones((DEGW,), jnp.float32)

        @pl.loop(0, DSTRIPE)
        def _(i):
            stripe_t[i, :] = jnp.zeros((DEGW,), jnp.float32)

        pltpu.sync_copy(stripe_t, deg_s.at[pl.ds(s * DSTRIPE, DSTRIPE)])
        plsc.subcore_barrier()

        w = c * NSUB + s

        def hit(t):
            pltpu.sync_copy(ei_hbm.at[t, 1], cidx)
            pltpu.sync_copy(ones_t, deg_s.at[cidx], add=True)

        @pl.loop(0, DCHUNKS)
        def _(k):
            hit(w + (NSC * NSUB) * k)

        @pl.when(w < DREM)
        def _():
            hit(DCHUNKS * NSC * NSUB + w)

        plsc.subcore_barrier()

        pltpu.sync_copy(deg_s.at[pl.ds(s * DSTRIPE, DSTRIPE)], stripe_t)
        lane0 = jnp.zeros((16,), jnp.int32)
        rowi = lax.iota(jnp.int32, 16)

        @pl.loop(0, DSTRIPE, step=16)
        def _(r):
            v = plsc.load_gather(stripe_t, [rowi + r, lane0])
            out1d[pl.ds(r, 16)] = v

        pltpu.sync_copy(out1d, deg_hbm.at[c, pl.ds(s * DSTRIPE, DSTRIPE)])

    return deg_kernel(ei_pack)


def _message_sc(g0, g1, ei_pack):
    @pl.kernel(
        out_type=[jax.ShapeDtypeStruct((N, EMB), jnp.float32),
                  jax.ShapeDtypeStruct((N, EMB), jnp.float32)],
        mesh=_sc_mesh(),
        scratch_types=[
            pltpu.VMEM_SHARED((N, EMB), jnp.float32),
            pltpu.VMEM((2, MCH), jnp.int32),
            pltpu.VMEM((2, MCH), jnp.int32),
            pltpu.VMEM((MCH, EMB), jnp.float32),
            pltpu.VMEM((MCH, EMB), jnp.float32),
            pltpu.SemaphoreType.DMA,
            pltpu.SemaphoreType.DMA,
            pltpu.SemaphoreType.DMA,
            pltpu.SemaphoreType.DMA,
        ],
        compiler_params=_SC_PARAMS,
    )
    def msg_kernel(g0_hbm, g1_hbm, ei_hbm, a0_hbm, a1_hbm,
                   acc_s, ib0, ib1, msg0, msg1, is0, is1, gs0, gs1):
        c = lax.axis_index("c")
        s = lax.axis_index("s")

        def run(g_hbm, a_hbm):
            stripe = pl.ds(s * MSTRIPE, MSTRIPE)
            pltpu.sync_copy(g_hbm.at[stripe], acc_s.at[stripe])
            plsc.subcore_barrier()

            bufs = ((ib0, msg0, is0, gs0), (ib1, msg1, is1, gs1))
            t0 = s * MSPANC

            def start_idx(j, b):
                ib, _, isem, _ = bufs[b]
                pltpu.async_copy(ei_hbm.at[t0 + j], ib, isem)

            def start_gather(j, b):
                ib, msg, isem, gsem = bufs[b]
                pltpu.make_async_copy(ei_hbm.at[t0 + j], ib, isem).wait()
                pltpu.async_copy(g_hbm.at[ib.at[0]], msg, gsem)

            def drain(b):
                ib, msg, _, gsem = bufs[b]
                pltpu.make_async_copy(g_hbm.at[ib.at[0]], msg, gsem).wait()
                pltpu.sync_copy(msg, acc_s.at[ib.at[1]], add=True)

            start_idx(0, 0)
            start_idx(1, 1)

            @pl.loop(0, (MSPANC - 1) // 2)
            def _(i):
                start_gather(2 * i, 0)
                start_gather(2 * i + 1, 1)
                drain(0)
                start_idx(2 * i + 2, 0)
                drain(1)

                @pl.when(2 * i + 3 < MSPANC)
                def _():
                    start_idx(2 * i + 3, 1)

            start_gather(MSPANC - 1, 0)
            drain(0)

            plsc.subcore_barrier()
            pltpu.sync_copy(acc_s.at[stripe], a_hbm.at[stripe])

        @pl.when(c == 0)
        def _():
            run(g0_hbm, a0_hbm)

        @pl.when(c == 1)
        def _():
            run(g1_hbm, a1_hbm)

    return msg_kernel(g0, g1, ei_pack)


def _fold(gamma, beta, s1, s2):
    m = s1 / N
    v = s2 / N - m * m
    s = gamma * lax.rsqrt(v + EPS)
    return s, beta - m * s


def kernel(high_dim_features, low_dim_features, edge_index,
           bn_low_g, bn_low_b, bn_high_g, bn_high_b,
           W_low, b_low, mlp_low_g, mlp_low_b,
           W_high, b_high, mlp_high_g, mlp_high_b,
           W_gcn, b_gcn, W_cls, b_cls):
    ei_pack = edge_index.reshape(2, NCHT, MCH).transpose(1, 0, 2)

    degp = _degree_sc(ei_pack)
    deg3 = (degp[0, :N] + degp[1, :N]).reshape(NBLK, 1, BN_ROWS)

    sh, sl = pl.pallas_call(
        _stats_body,
        grid=(NBLK,),
        in_specs=[
            pl.BlockSpec((BN_ROWS, HD), lambda i: (i, 0)),
            pl.BlockSpec((BN_ROWS, LD), lambda i: (i, 0)),
        ],
        out_specs=[
            pl.BlockSpec((8, HD), lambda i: (0, 0)),
            pl.BlockSpec((8, LD), lambda i: (0, 0)),
        ],
        out_shape=[
            jax.ShapeDtypeStruct((8, HD), jnp.float32),
            jax.ShapeDtypeStruct((8, LD), jnp.float32),
        ],
    )(high_dim_features, low_dim_features)

    s_hi, t_hi = _fold(bn_high_g, bn_high_b, sh[0], sh[1])
    s_lo, t_lo = _fold(bn_low_g, bn_low_b, sl[0], sl[1])
    Wh = s_hi[:, None] * W_high
    bh = (t_hi @ W_high + b_high)[None, :]
    Wl = s_lo[:, None] * W_low
    bl = (t_lo @ W_low + b_low)[None, :]

    y, sy = pl.pallas_call(
        _embed_body,
        grid=(NBLK,),
        in_specs=[
            pl.BlockSpec((BN_ROWS, HD), lambda i: (i, 0)),
            pl.BlockSpec((BN_ROWS, LD), lambda i: (i, 0)),
            pl.BlockSpec((HD, EMB), lambda i: (0, 0)),
            pl.BlockSpec((1, EMB), lambda i: (0, 0)),
            pl.BlockSpec((LD, EMB), lambda i: (0, 0)),
            pl.BlockSpec((1, EMB), lambda i: (0, 0)),
        ],
        out_specs=[
            pl.BlockSpec((BN_ROWS, HID), lambda i: (i, 0)),
            pl.BlockSpec((8, HID), lambda i: (0, 0)),
        ],
        out_shape=[
            jax.ShapeDtypeStruct((N, HID), jnp.float32),
            jax.ShapeDtypeStruct((8, HID), jnp.float32),
        ],
    )(high_dim_features, low_dim_features, Wh, bh, Wl, bl)

    gy = jnp.concatenate([mlp_high_g, mlp_low_g])
    by = jnp.concatenate([mlp_high_b, mlp_low_b])
    s_y, t_y = _fold(gy, by, sy[0], sy[1])
    W2 = s_y[:, None] * W_gcn
    c2 = (t_y @ W_gcn)[None, :]

    g0, g1 = pl.pallas_call(
        _project_body,
        grid=(NBLK,),
        in_specs=[
            pl.BlockSpec((BN_ROWS, HID), lambda i: (i, 0)),
            pl.BlockSpec((HID, HID), lambda i: (0, 0)),
            pl.BlockSpec((1, HID), lambda i: (0, 0)),
            pl.BlockSpec((1, 1, BN_ROWS), lambda i: (i, 0, 0)),
        ],
        out_specs=[
            pl.BlockSpec((BN_ROWS, EMB), lambda i: (i, 0)),
            pl.BlockSpec((BN_ROWS, EMB), lambda i: (i, 0)),
        ],
        out_shape=[
            jax.ShapeDtypeStruct((N, EMB), jnp.float32),
            jax.ShapeDtypeStruct((N, EMB), jnp.float32),
        ],
    )(y, W2, c2, deg3)

    acc0, acc1 = _message_sc(g0, g1, ei_pack)

    out = pl.pallas_call(
        _final_body,
        grid=(NBLK,),
        in_specs=[
            pl.BlockSpec((BN_ROWS, EMB), lambda i: (i, 0)),
            pl.BlockSpec((BN_ROWS, EMB), lambda i: (i, 0)),
            pl.BlockSpec((1, 1, BN_ROWS), lambda i: (i, 0, 0)),
            pl.BlockSpec((1, HID), lambda i: (0, 0)),
            pl.BlockSpec((HID, OUT), lambda i: (0, 0)),
            pl.BlockSpec((1, OUT), lambda i: (0, 0)),
        ],
        out_specs=pl.BlockSpec((BN_ROWS, OUT), lambda i: (i, 0)),
        out_shape=jax.ShapeDtypeStruct((N, OUT), jnp.float32),
    )(acc0, acc1, deg3, b_gcn[None, :], W_cls, b_cls[None, :])

    return out


# speedup vs baseline: 43.3509x; 1.1785x over previous
import jax
import jax.numpy as jnp
from jax import lax
from jax.experimental import pallas as pl
from jax.experimental.pallas import tpu as pltpu
from jax.experimental.pallas import tpu_sc as plsc

N = 50000
E = 800000
HD = 512
LD = 16
EMB = 32
HID = 64
OUT = 40
EPS = 1e-5

BN_ROWS = 2000
NBLK = N // BN_ROWS

NSC = 2
NSUB = 16
DEGW = 16
NP = 50176
DSTRIPE = NP // NSUB
DCH = 1000
DSPAN = E // (NSC * NSUB)

MCH = 400
MSPANC = E // (NSUB * MCH)
MSTRIPE = N // NSUB
ZROWS = 125


def _stats_body(hi_ref, lo_ref, sh_ref, sl_ref):
    i = pl.program_id(0)

    @pl.when(i == 0)
    def _():
        sh_ref[...] = jnp.zeros_like(sh_ref)
        sl_ref[...] = jnp.zeros_like(sl_ref)

    hi = hi_ref[...]
    lo = lo_ref[...]
    sh_ref[0, :] += jnp.sum(hi, axis=0)
    sh_ref[1, :] += jnp.sum(hi * hi, axis=0)
    sl_ref[0, :] += jnp.sum(lo, axis=0)
    sl_ref[1, :] += jnp.sum(lo * lo, axis=0)


def _embed_body(hi_ref, lo_ref, wh_ref, bh_ref, wl_ref, bl_ref, y_ref, sy_ref):
    i = pl.program_id(0)

    @pl.when(i == 0)
    def _():
        sy_ref[...] = jnp.zeros_like(sy_ref)

    yh = jnp.maximum(
        jnp.dot(hi_ref[...], wh_ref[...], preferred_element_type=jnp.float32)
        + bh_ref[0, :], 0.0)
    yl = jnp.maximum(
        jnp.dot(lo_ref[...], wl_ref[...], preferred_element_type=jnp.float32)
        + bl_ref[0, :], 0.0)
    y = jnp.concatenate([yh, yl], axis=1)
    y_ref[...] = y
    sy_ref[0, :] += jnp.sum(y, axis=0)
    sy_ref[1, :] += jnp.sum(y * y, axis=0)


def _dinv_col(deg_ref):
    d = jnp.reshape(deg_ref[0, 0, :], (BN_ROWS, 1))
    return lax.rsqrt(1.0 + d)


def _project_body(y_ref, w2_ref, c2_ref, deg_ref, g_ref):
    h = jnp.dot(y_ref[...], w2_ref[...], preferred_element_type=jnp.float32)
    h = h + c2_ref[0, :]
    g = h * _dinv_col(deg_ref)
    g_ref[...] = jnp.concatenate([g, jnp.zeros_like(g)], axis=1)


def _final_body(a0_ref, a1_ref, g_ref, deg_ref, bg_ref, wc_ref, bc_ref, o_ref):
    acc = jnp.concatenate([a0_ref[...], a1_ref[...]], axis=1) + g_ref[:, :HID]
    z = jnp.tanh(_dinv_col(deg_ref) * acc + bg_ref[0, :])
    logits = jnp.dot(z, wc_ref[...], preferred_element_type=jnp.float32)
    logits = logits + bc_ref[0, :]
    m = jnp.max(logits, axis=1, keepdims=True)
    lse = m + jnp.log(jnp.sum(jnp.exp(logits - m), axis=1, keepdims=True))
    o_ref[...] = logits - lse


def _sc_mesh():
    return plsc.VectorSubcoreMesh(core_axis_name="c", subcore_axis_name="s")


_SC_PARAMS_NL = pltpu.CompilerParams(use_tc_tiling_on_sc=False,
                                     needs_layout_passes=False)


def _degree_sc(ei_flat):
    @pl.kernel(
        out_type=jax.ShapeDtypeStruct((NSC, NP), jnp.float32),
        mesh=_sc_mesh(),
        scratch_types=[
            pltpu.VMEM_SHARED((NP, DEGW), jnp.float32),
            pltpu.VMEM((DCH,), jnp.int32),
            pltpu.VMEM((DCH, DEGW), jnp.float32),
            pltpu.VMEM((DSTRIPE, DEGW), jnp.float32),
            pltpu.VMEM((DSTRIPE,), jnp.float32),
        ],
        compiler_params=_SC_PARAMS_NL,
    )
    def deg_kernel(ei_hbm, deg_hbm, deg_s, cidx, ones_t, stripe_t, out1d):
        c = lax.axis_index("c")
        s = lax.axis_index("s")

        @pl.loop(0, DCH)
        def _(i):
            ones_t[i, :] = jnp.ones((DEGW,), jnp.float32)

        @pl.loop(0, DSTRIPE)
        def _(i):
            stripe_t[i, :] = jnp.zeros((DEGW,), jnp.float32)

        pltpu.sync_copy(stripe_t, deg_s.at[pl.ds(s * DSTRIPE, DSTRIPE)])
        plsc.subcore_barrier()

        w = c * NSUB + s

        @pl.loop(0, DSPAN // DCH)
        def _(j):
            base = E + w * DSPAN + j * DCH
            pltpu.sync_copy(ei_hbm.at[pl.ds(base, DCH)], cidx)
            pltpu.sync_copy(ones_t, deg_s.at[cidx], add=True)

        plsc.subcore_barrier()

        pltpu.sync_copy(deg_s.at[pl.ds(s * DSTRIPE, DSTRIPE)], stripe_t)
        lane0 = jnp.zeros((16,), jnp.int32)
        rowi = lax.iota(jnp.int32, 16)

        @pl.loop(0, DSTRIPE, step=16)
        def _(r):
            v = plsc.load_gather(stripe_t, [rowi + r, lane0])
            out1d[pl.ds(r, 16)] = v

        pltpu.sync_copy(out1d, deg_hbm.at[c, pl.ds(s * DSTRIPE, DSTRIPE)])

    return deg_kernel(ei_flat)


def _message_sc(g128, ei_flat):
    @pl.kernel(
        out_type=[jax.ShapeDtypeStruct((N, EMB), jnp.float32),
                  jax.ShapeDtypeStruct((N, EMB), jnp.float32)],
        mesh=_sc_mesh(),
        scratch_types=[
            pltpu.VMEM_SHARED((N, EMB), jnp.float32),
            pltpu.VMEM((2, MCH), jnp.int32),
            pltpu.VMEM((2, MCH), jnp.int32),
            pltpu.VMEM((MCH, EMB), jnp.float32),
            pltpu.VMEM((MCH, EMB), jnp.float32),
            pltpu.SemaphoreType.DMA,
            pltpu.SemaphoreType.DMA,
            pltpu.SemaphoreType.DMA,
            pltpu.SemaphoreType.DMA,
        ],
        compiler_params=_SC_PARAMS_NL,
    )
    def msg_kernel(gview, ei_hbm, a0_hbm, a1_hbm,
                   acc_s, ib0, ib1, msg0, msg1,
                   is0, is1, gs0, gs1):
        c = lax.axis_index("c")
        s = lax.axis_index("s")

        @pl.loop(0, ZROWS)
        def _(i):
            msg0[i, pl.ds(0, 16)] = jnp.zeros((16,), jnp.float32)
            msg0[i, pl.ds(16, 16)] = jnp.zeros((16,), jnp.float32)

        @pl.loop(0, MSTRIPE // ZROWS)
        def _(k):
            pltpu.sync_copy(
                msg0.at[pl.ds(0, ZROWS)],
                acc_s.at[pl.ds(s * MSTRIPE + k * ZROWS, ZROWS)])

        plsc.subcore_barrier()

        def run(a_hbm):
            bufs = ((ib0, msg0, is0, gs0), (ib1, msg1, is1, gs1))
            base0 = s * MSPANC * MCH

            def idx_descs(j, b):
                ib = bufs[b][0]
                isem = bufs[b][2]
                base = base0 + j * MCH
                return (pltpu.make_async_copy(
                            ei_hbm.at[pl.ds(base, MCH)], ib.at[0], isem),
                        pltpu.make_async_copy(
                            ei_hbm.at[pl.ds(E + base, MCH)], ib.at[1], isem))

            def start_idx(j, b):
                d0, d1 = idx_descs(j, b)
                d0.start()
                d1.start()

            def start_gather(j, b):
                ib, msg, isem, gsem = bufs[b]
                d0, d1 = idx_descs(j, b)
                d0.wait()
                d1.wait()

                @pl.loop(0, MCH, step=16)
                def _(k):
                    v = ib[0, pl.ds(k, 16)]
                    ib[0, pl.ds(k, 16)] = v * 4 + c

                pltpu.async_copy(gview.at[ib.at[0]], msg, gsem)

            def drain(b):
                ib, msg, isem, gsem = bufs[b]
                pltpu.make_async_copy(gview.at[ib.at[0]], msg, gsem).wait()
                pltpu.sync_copy(msg, acc_s.at[ib.at[1]], add=True)

            start_idx(0, 0)
            start_idx(1, 1)

            @pl.loop(0, (MSPANC - 1) // 2)
            def _(i):
                start_gather(2 * i, 0)
                start_gather(2 * i + 1, 1)
                drain(0)
                start_idx(2 * i + 2, 0)
                drain(1)

                @pl.when(2 * i + 3 < MSPANC)
                def _():
                    start_idx(2 * i + 3, 1)

            start_gather(MSPANC - 1, 0)
            drain(0)

            plsc.subcore_barrier()
            stripe = pl.ds(s * MSTRIPE, MSTRIPE)
            pltpu.sync_copy(acc_s.at[stripe], a_hbm.at[stripe])

        @pl.when(c == 0)
        def _():
            run(a0_hbm)

        @pl.when(c == 1)
        def _():
            run(a1_hbm)

    return msg_kernel(g128.reshape(4 * N, EMB), ei_flat)


def _fold(gamma, beta, s1, s2):
    m = s1 / N
    v = s2 / N - m * m
    s = gamma * lax.rsqrt(v + EPS)
    return s, beta - m * s


def kernel(high_dim_features, low_dim_features, edge_index,
           bn_low_g, bn_low_b, bn_high_g, bn_high_b,
           W_low, b_low, mlp_low_g, mlp_low_b,
           W_high, b_high, mlp_high_g, mlp_high_b,
           W_gcn, b_gcn, W_cls, b_cls):
    ei_flat = edge_index.reshape(2 * E)

    degp = _degree_sc(ei_flat)
    deg3 = (degp[0, :N] + degp[1, :N]).reshape(NBLK, 1, BN_ROWS)

    sh, sl = pl.pallas_call(
        _stats_body,
        grid=(NBLK,),
        in_specs=[
            pl.BlockSpec((BN_ROWS, HD), lambda i: (i, 0)),
            pl.BlockSpec((BN_ROWS, LD), lambda i: (i, 0)),
        ],
        out_specs=[
            pl.BlockSpec((8, HD), lambda i: (0, 0)),
            pl.BlockSpec((8, LD), lambda i: (0, 0)),
        ],
        out_shape=[
            jax.ShapeDtypeStruct((8, HD), jnp.float32),
            jax.ShapeDtypeStruct((8, LD), jnp.float32),
        ],
    )(high_dim_features, low_dim_features)

    s_hi, t_hi = _fold(bn_high_g, bn_high_b, sh[0], sh[1])
    s_lo, t_lo = _fold(bn_low_g, bn_low_b, sl[0], sl[1])
    Wh = s_hi[:, None] * W_high
    bh = (t_hi @ W_high + b_high)[None, :]
    Wl = s_lo[:, None] * W_low
    bl = (t_lo @ W_low + b_low)[None, :]

    y, sy = pl.pallas_call(
        _embed_body,
        grid=(NBLK,),
        in_specs=[
            pl.BlockSpec((BN_ROWS, HD), lambda i: (i, 0)),
            pl.BlockSpec((BN_ROWS, LD), lambda i: (i, 0)),
            pl.BlockSpec((HD, EMB), lambda i: (0, 0)),
            pl.BlockSpec((1, EMB), lambda i: (0, 0)),
            pl.BlockSpec((LD, EMB), lambda i: (0, 0)),
            pl.BlockSpec((1, EMB), lambda i: (0, 0)),
        ],
        out_specs=[
            pl.BlockSpec((BN_ROWS, HID), lambda i: (i, 0)),
            pl.BlockSpec((8, HID), lambda i: (0, 0)),
        ],
        out_shape=[
            jax.ShapeDtypeStruct((N, HID), jnp.float32),
            jax.ShapeDtypeStruct((8, HID), jnp.float32),
        ],
    )(high_dim_features, low_dim_features, Wh, bh, Wl, bl)

    gy = jnp.concatenate([mlp_high_g, mlp_low_g])
    by = jnp.concatenate([mlp_high_b, mlp_low_b])
    s_y, t_y = _fold(gy, by, sy[0], sy[1])
    W2 = s_y[:, None] * W_gcn
    c2 = (t_y @ W_gcn)[None, :]

    g128 = pl.pallas_call(
        _project_body,
        grid=(NBLK,),
        in_specs=[
            pl.BlockSpec((BN_ROWS, HID), lambda i: (i, 0)),
            pl.BlockSpec((HID, HID), lambda i: (0, 0)),
            pl.BlockSpec((1, HID), lambda i: (0, 0)),
            pl.BlockSpec((1, 1, BN_ROWS), lambda i: (i, 0, 0)),
        ],
        out_specs=pl.BlockSpec((BN_ROWS, 2 * HID), lambda i: (i, 0)),
        out_shape=jax.ShapeDtypeStruct((N, 2 * HID), jnp.float32),
    )(y, W2, c2, deg3)

    acc0, acc1 = _message_sc(g128, ei_flat)
    acc0v = acc0.reshape(NBLK, BN_ROWS // 4, 4 * EMB)
    acc1v = acc1.reshape(NBLK, BN_ROWS // 4, 4 * EMB)

    out = pl.pallas_call(
        _final_body,
        grid=(NBLK,),
        in_specs=[
            pl.BlockSpec((BN_ROWS, EMB), lambda i: (i, 0)),
            pl.BlockSpec((BN_ROWS, EMB), lambda i: (i, 0)),
            pl.BlockSpec((BN_ROWS, 2 * HID), lambda i: (i, 0)),
            pl.BlockSpec((1, 1, BN_ROWS), lambda i: (i, 0, 0)),
            pl.BlockSpec((1, HID), lambda i: (0, 0)),
            pl.BlockSpec((HID, OUT), lambda i: (0, 0)),
            pl.BlockSpec((1, OUT), lambda i: (0, 0)),
        ],
        out_specs=pl.BlockSpec((BN_ROWS, OUT), lambda i: (i, 0)),
        out_shape=jax.ShapeDtypeStruct((N, OUT), jnp.float32),
    )(acc0, acc1, g128, deg3, b_gcn[None, :], W_cls, b_cls[None, :])

    return out


# speedup vs baseline: 49.9206x; 1.1515x over previous
import jax
import jax.numpy as jnp
from jax import lax
from jax.experimental import pallas as pl
from jax.experimental.pallas import tpu as pltpu
from jax.experimental.pallas import tpu_sc as plsc

N = 50000
E = 800000
HD = 512
LD = 16
EMB = 32
HID = 64
OUT = 40
EPS = 1e-5

BN_ROWS = 5000
NBLK = N // BN_ROWS

NSC = 2
NSUB = 16
DEGW = 16
NP = 50176
DSTRIPE = NP // NSUB
DCH = 1000
DSPAN = E // (NSC * NSUB)

MCH = 400
MSPANC = E // (NSUB * MCH)
MSTRIPE = N // NSUB
ZROWS = 125


def _stats_body(hi_ref, lo_ref, sh_ref, sl_ref):
    i = pl.program_id(0)

    @pl.when(i == 0)
    def _():
        sh_ref[...] = jnp.zeros_like(sh_ref)
        sl_ref[...] = jnp.zeros_like(sl_ref)

    hi = hi_ref[...]
    lo = lo_ref[...]
    sh_ref[0, :] += jnp.sum(hi, axis=0)
    sh_ref[1, :] += jnp.sum(hi * hi, axis=0)
    sl_ref[0, :] += jnp.sum(lo, axis=0)
    sl_ref[1, :] += jnp.sum(lo * lo, axis=0)


def _embed_body(hi_ref, lo_ref, wh_ref, bh_ref, wl_ref, bl_ref, y_ref, sy_ref):
    i = pl.program_id(0)

    @pl.when(i == 0)
    def _():
        sy_ref[...] = jnp.zeros_like(sy_ref)

    yh = jnp.maximum(
        jnp.dot(hi_ref[...], wh_ref[...], preferred_element_type=jnp.float32)
        + bh_ref[0, :], 0.0)
    yl = jnp.maximum(
        jnp.dot(lo_ref[...], wl_ref[...], preferred_element_type=jnp.float32)
        + bl_ref[0, :], 0.0)
    y = jnp.concatenate([yh, yl], axis=1)
    y_ref[...] = y
    sy_ref[0, :] += jnp.sum(y, axis=0)
    sy_ref[1, :] += jnp.sum(y * y, axis=0)


def _dinv_col(deg_ref):
    d = jnp.reshape(deg_ref[0, 0, :], (BN_ROWS, 1))
    return lax.rsqrt(1.0 + d)


def _project_body(y_ref, sy_ref, ty_ref, deg_ref, g_ref):
    u = y_ref[...] * sy_ref[0, :] + ty_ref[0, :]
    g = u * _dinv_col(deg_ref)
    g_ref[...] = jnp.concatenate([g, jnp.zeros_like(g)], axis=1)


def _final_body(a_ref, g_ref, deg_ref, w2_ref, bg_ref, wc_ref, bc_ref, o_ref):
    t = _dinv_col(deg_ref) * (a_ref[:, :HID] + g_ref[:, :HID])
    z = jnp.tanh(
        jnp.dot(t, w2_ref[...], preferred_element_type=jnp.float32)
        + bg_ref[0, :])
    logits = jnp.dot(z, wc_ref[...], preferred_element_type=jnp.float32)
    logits = logits + bc_ref[0, :]
    m = jnp.max(logits, axis=1, keepdims=True)
    lse = m + jnp.log(jnp.sum(jnp.exp(logits - m), axis=1, keepdims=True))
    o_ref[...] = logits - lse


def _sc_mesh():
    return plsc.VectorSubcoreMesh(core_axis_name="c", subcore_axis_name="s")


_SC_PARAMS_NL = pltpu.CompilerParams(use_tc_tiling_on_sc=False,
                                     needs_layout_passes=False)


def _degree_sc(ei_flat):
    @pl.kernel(
        out_type=jax.ShapeDtypeStruct((NSC, NP), jnp.float32),
        mesh=_sc_mesh(),
        scratch_types=[
            pltpu.VMEM_SHARED((NP, DEGW), jnp.float32),
            pltpu.VMEM((DCH,), jnp.int32),
            pltpu.VMEM((DCH, DEGW), jnp.float32),
            pltpu.VMEM((DSTRIPE, DEGW), jnp.float32),
            pltpu.VMEM((DSTRIPE,), jnp.float32),
        ],
        compiler_params=_SC_PARAMS_NL,
    )
    def deg_kernel(ei_hbm, deg_hbm, deg_s, cidx, ones_t, stripe_t, out1d):
        c = lax.axis_index("c")
        s = lax.axis_index("s")

        @pl.loop(0, DCH)
        def _(i):
            ones_t[i, :] = jnp.ones((DEGW,), jnp.float32)

        @pl.loop(0, DSTRIPE)
        def _(i):
            stripe_t[i, :] = jnp.zeros((DEGW,), jnp.float32)

        pltpu.sync_copy(stripe_t, deg_s.at[pl.ds(s * DSTRIPE, DSTRIPE)])
        plsc.subcore_barrier()

        w = c * NSUB + s

        @pl.loop(0, DSPAN // DCH)
        def _(j):
            base = E + w * DSPAN + j * DCH
            pltpu.sync_copy(ei_hbm.at[pl.ds(base, DCH)], cidx)
            pltpu.sync_copy(ones_t, deg_s.at[cidx], add=True)

        plsc.subcore_barrier()

        pltpu.sync_copy(deg_s.at[pl.ds(s * DSTRIPE, DSTRIPE)], stripe_t)
        lane0 = jnp.zeros((16,), jnp.int32)
        rowi = lax.iota(jnp.int32, 16)

        @pl.loop(0, DSTRIPE, step=16)
        def _(r):
            v = plsc.load_gather(stripe_t, [rowi + r, lane0])
            out1d[pl.ds(r, 16)] = v

        pltpu.sync_copy(out1d, deg_hbm.at[c, pl.ds(s * DSTRIPE, DSTRIPE)])

    return deg_kernel(ei_flat)


def _message_sc(g128, ei_flat):
    @pl.kernel(
        out_type=jax.ShapeDtypeStruct((N, 4 * EMB), jnp.float32),
        mesh=_sc_mesh(),
        scratch_types=[
            pltpu.VMEM_SHARED((N, EMB), jnp.float32),
            pltpu.VMEM((2, MCH), jnp.int32),
            pltpu.VMEM((2, MCH), jnp.int32),
            pltpu.VMEM((MCH, EMB), jnp.float32),
            pltpu.VMEM((MCH, EMB), jnp.float32),
            pltpu.SemaphoreType.DMA,
            pltpu.SemaphoreType.DMA,
            pltpu.SemaphoreType.DMA,
            pltpu.SemaphoreType.DMA,
        ],
        compiler_params=_SC_PARAMS_NL,
    )
    def msg_kernel(gview, ei_hbm, a_hbm,
                   acc_s, ib0, ib1, msg0, msg1,
                   is0, is1, gs0, gs1):
        c = lax.axis_index("c")
        s = lax.axis_index("s")

        @pl.loop(0, ZROWS)
        def _(i):
            msg0[i, pl.ds(0, 16)] = jnp.zeros((16,), jnp.float32)
            msg0[i, pl.ds(16, 16)] = jnp.zeros((16,), jnp.float32)

        @pl.loop(0, MSTRIPE // ZROWS)
        def _(k):
            pltpu.sync_copy(
                msg0.at[pl.ds(0, ZROWS)],
                acc_s.at[pl.ds(s * MSTRIPE + k * ZROWS, ZROWS)])

        plsc.subcore_barrier()

        def run():
            bufs = ((ib0, msg0, is0, gs0), (ib1, msg1, is1, gs1))
            base0 = s * MSPANC * MCH

            def idx_descs(j, b):
                ib = bufs[b][0]
                isem = bufs[b][2]
                base = base0 + j * MCH
                return (pltpu.make_async_copy(
                            ei_hbm.at[pl.ds(base, MCH)], ib.at[0], isem),
                        pltpu.make_async_copy(
                            ei_hbm.at[pl.ds(E + base, MCH)], ib.at[1], isem))

            def start_idx(j, b):
                d0, d1 = idx_descs(j, b)
                d0.start()
                d1.start()

            def start_gather(j, b):
                ib, msg, isem, gsem = bufs[b]
                d0, d1 = idx_descs(j, b)
                d0.wait()
                d1.wait()

                @pl.loop(0, MCH, step=16)
                def _(k):
                    v = ib[0, pl.ds(k, 16)]
                    ib[0, pl.ds(k, 16)] = v * 4 + c

                pltpu.async_copy(gview.at[ib.at[0]], msg, gsem)

            def drain(b):
                ib, msg, isem, gsem = bufs[b]
                pltpu.make_async_copy(gview.at[ib.at[0]], msg, gsem).wait()
                pltpu.sync_copy(msg, acc_s.at[ib.at[1]], add=True)

            start_idx(0, 0)
            start_idx(1, 1)

            @pl.loop(0, (MSPANC - 1) // 2)
            def _(i):
                start_gather(2 * i, 0)
                start_gather(2 * i + 1, 1)
                drain(0)
                start_idx(2 * i + 2, 0)
                drain(1)

                @pl.when(2 * i + 3 < MSPANC)
                def _():
                    start_idx(2 * i + 3, 1)

            start_gather(MSPANC - 1, 0)
            drain(0)

            plsc.subcore_barrier()
            stripe = pl.ds(s * MSTRIPE, MSTRIPE)
            pltpu.sync_copy(acc_s.at[stripe],
                            a_hbm.at[stripe, pl.ds(EMB * c, EMB)])

        run()

    return msg_kernel(g128.reshape(4 * N, EMB), ei_flat)


def _fold(gamma, beta, s1, s2):
    m = s1 / N
    v = s2 / N - m * m
    s = gamma * lax.rsqrt(v + EPS)
    return s, beta - m * s


def kernel(high_dim_features, low_dim_features, edge_index,
           bn_low_g, bn_low_b, bn_high_g, bn_high_b,
           W_low, b_low, mlp_low_g, mlp_low_b,
           W_high, b_high, mlp_high_g, mlp_high_b,
           W_gcn, b_gcn, W_cls, b_cls):
    ei_flat = edge_index.reshape(2 * E)

    degp = _degree_sc(ei_flat)
    deg3 = (degp[0, :N] + degp[1, :N]).reshape(NBLK, 1, BN_ROWS)

    sh, sl = pl.pallas_call(
        _stats_body,
        grid=(NBLK,),
        in_specs=[
            pl.BlockSpec((BN_ROWS, HD), lambda i: (i, 0)),
            pl.BlockSpec((BN_ROWS, LD), lambda i: (i, 0)),
        ],
        out_specs=[
            pl.BlockSpec((8, HD), lambda i: (0, 0)),
            pl.BlockSpec((8, LD), lambda i: (0, 0)),
        ],
        out_shape=[
            jax.ShapeDtypeStruct((8, HD), jnp.float32),
            jax.ShapeDtypeStruct((8, LD), jnp.float32),
        ],
    )(high_dim_features, low_dim_features)

    s_hi, t_hi = _fold(bn_high_g, bn_high_b, sh[0], sh[1])
    s_lo, t_lo = _fold(bn_low_g, bn_low_b, sl[0], sl[1])
    Wh = s_hi[:, None] * W_high
    bh = (t_hi @ W_high + b_high)[None, :]
    Wl = s_lo[:, None] * W_low
    bl = (t_lo @ W_low + b_low)[None, :]

    y, sy = pl.pallas_call(
        _embed_body,
        grid=(NBLK,),
        in_specs=[
            pl.BlockSpec((BN_ROWS, HD), lambda i: (i, 0)),
            pl.BlockSpec((BN_ROWS, LD), lambda i: (i, 0)),
            pl.BlockSpec((HD, EMB), lambda i: (0, 0)),
            pl.BlockSpec((1, EMB), lambda i: (0, 0)),
            pl.BlockSpec((LD, EMB), lambda i: (0, 0)),
            pl.BlockSpec((1, EMB), lambda i: (0, 0)),
        ],
        out_specs=[
            pl.BlockSpec((BN_ROWS, HID), lambda i: (i, 0)),
            pl.BlockSpec((8, HID), lambda i: (0, 0)),
        ],
        out_shape=[
            jax.ShapeDtypeStruct((N, HID), jnp.float32),
            jax.ShapeDtypeStruct((8, HID), jnp.float32),
        ],
    )(high_dim_features, low_dim_features, Wh, bh, Wl, bl)

    gy = jnp.concatenate([mlp_high_g, mlp_low_g])
    by = jnp.concatenate([mlp_high_b, mlp_low_b])
    s_y, t_y = _fold(gy, by, sy[0], sy[1])

    g128 = pl.pallas_call(
        _project_body,
        grid=(NBLK,),
        in_specs=[
            pl.BlockSpec((BN_ROWS, HID), lambda i: (i, 0)),
            pl.BlockSpec((1, HID), lambda i: (0, 0)),
            pl.BlockSpec((1, HID), lambda i: (0, 0)),
            pl.BlockSpec((1, 1, BN_ROWS), lambda i: (i, 0, 0)),
        ],
        out_specs=pl.BlockSpec((BN_ROWS, 2 * HID), lambda i: (i, 0)),
        out_shape=jax.ShapeDtypeStruct((N, 2 * HID), jnp.float32),
    )(y, s_y[None, :], t_y[None, :], deg3)

    acc128 = _message_sc(g128, ei_flat)

    out = pl.pallas_call(
        _final_body,
        grid=(NBLK,),
        in_specs=[
            pl.BlockSpec((BN_ROWS, 4 * EMB), lambda i: (i, 0)),
            pl.BlockSpec((BN_ROWS, 2 * HID), lambda i: (i, 0)),
            pl.BlockSpec((1, 1, BN_ROWS), lambda i: (i, 0, 0)),
            pl.BlockSpec((HID, HID), lambda i: (0, 0)),
            pl.BlockSpec((1, HID), lambda i: (0, 0)),
            pl.BlockSpec((HID, OUT), lambda i: (0, 0)),
            pl.BlockSpec((1, OUT), lambda i: (0, 0)),
        ],
        out_specs=pl.BlockSpec((BN_ROWS, OUT), lambda i: (i, 0)),
        out_shape=jax.ShapeDtypeStruct((N, OUT), jnp.float32),
    )(acc128, g128, deg3, W_gcn, b_gcn[None, :], W_cls, b_cls[None, :])

    return out


# speedup vs baseline: 52.3277x; 1.0482x over previous
import jax
import jax.numpy as jnp
from jax import lax
from jax.experimental import pallas as pl
from jax.experimental.pallas import tpu as pltpu
from jax.experimental.pallas import tpu_sc as plsc

N = 50000
E = 800000
HD = 512
LD = 16
EMB = 32
HID = 64
OUT = 40
EPS = 1e-5

BN_ROWS = 5000
NBLK = N // BN_ROWS

NSC = 2
NSUB = 16
DEGW = 16
NP = 50176
DSTRIPE = NP // NSUB
DCH = 1000
DSPAN = E // (NSC * NSUB)

MCH = 400
MSPANC = E // (NSUB * MCH)
MSTRIPE = N // NSUB
ZROWS = 125


def _stats_body(hi_ref, lo_ref, sh_ref, sl_ref):
    i = pl.program_id(0)

    @pl.when(i == 0)
    def _():
        sh_ref[...] = jnp.zeros_like(sh_ref)
        sl_ref[...] = jnp.zeros_like(sl_ref)

    hi = hi_ref[...]
    lo = lo_ref[:, 0, 0, :]
    sh_ref[0, :] += jnp.sum(hi, axis=0)
    sh_ref[1, :] += jnp.sum(hi * hi, axis=0)
    sl_ref[0, :] += jnp.sum(lo, axis=1)
    sl_ref[1, :] += jnp.sum(lo * lo, axis=1)


def _embed_body(hi_ref, lo_ref, wh_ref, bh_ref, wl_ref, bl_ref, y_ref, sy_ref):
    i = pl.program_id(0)

    @pl.when(i == 0)
    def _():
        sy_ref[...] = jnp.zeros_like(sy_ref)

    yh = jnp.maximum(
        jnp.dot(hi_ref[...], wh_ref[...], preferred_element_type=jnp.float32)
        + bh_ref[0, :], 0.0)
    lo = lo_ref[:, 0, 0, :]
    yl = jnp.maximum(
        lax.dot_general(lo, wl_ref[...], (((0,), (0,)), ((), ())),
                        preferred_element_type=jnp.float32)
        + bl_ref[0, :], 0.0)
    y = jnp.concatenate([yh, yl], axis=1)
    y_ref[...] = y
    sy_ref[0, :] += jnp.sum(y, axis=0)
    sy_ref[1, :] += jnp.sum(y * y, axis=0)


def _dinv_col(deg_ref):
    d = jnp.reshape(deg_ref[0, 0, :], (BN_ROWS, 1))
    return lax.rsqrt(1.0 + d)


def _project_body(y_ref, sy_ref, ty_ref, deg_ref, g_ref):
    u = y_ref[...] * sy_ref[0, :] + ty_ref[0, :]
    g = u * _dinv_col(deg_ref)
    g_ref[...] = jnp.concatenate([g, jnp.zeros_like(g)], axis=1)


def _final_body(a_ref, g_ref, deg_ref, w2_ref, bg_ref, wc_ref, bc_ref, o_ref):
    t = _dinv_col(deg_ref) * (a_ref[:, :HID] + g_ref[:, :HID])
    z = jnp.tanh(
        jnp.dot(t, w2_ref[...], preferred_element_type=jnp.float32)
        + bg_ref[0, :])
    logits = jnp.dot(z, wc_ref[...], preferred_element_type=jnp.float32)
    logits = logits + bc_ref[0, :]
    m = jnp.max(logits, axis=1, keepdims=True)
    lse = m + jnp.log(jnp.sum(jnp.exp(logits - m), axis=1, keepdims=True))
    o_ref[...] = logits - lse


def _sc_mesh():
    return plsc.VectorSubcoreMesh(core_axis_name="c", subcore_axis_name="s")


_SC_PARAMS_NL = pltpu.CompilerParams(use_tc_tiling_on_sc=False,
                                     needs_layout_passes=False)


def _degree_sc(ei_flat):
    @pl.kernel(
        out_type=jax.ShapeDtypeStruct((NSC, NP), jnp.float32),
        mesh=_sc_mesh(),
        scratch_types=[
            pltpu.VMEM_SHARED((NP, DEGW), jnp.float32),
            pltpu.VMEM((DCH,), jnp.int32),
            pltpu.VMEM((DCH, DEGW), jnp.float32),
            pltpu.VMEM((DSTRIPE, DEGW), jnp.float32),
            pltpu.VMEM((DSTRIPE,), jnp.float32),
        ],
        compiler_params=_SC_PARAMS_NL,
    )
    def deg_kernel(ei_hbm, deg_hbm, deg_s, cidx, ones_t, stripe_t, out1d):
        c = lax.axis_index("c")
        s = lax.axis_index("s")

        @pl.loop(0, DCH)
        def _(i):
            ones_t[i, :] = jnp.ones((DEGW,), jnp.float32)

        @pl.loop(0, DSTRIPE)
        def _(i):
            stripe_t[i, :] = jnp.zeros((DEGW,), jnp.float32)

        pltpu.sync_copy(stripe_t, deg_s.at[pl.ds(s * DSTRIPE, DSTRIPE)])
        plsc.subcore_barrier()

        w = c * NSUB + s

        @pl.loop(0, DSPAN // DCH)
        def _(j):
            base = E + w * DSPAN + j * DCH
            pltpu.sync_copy(ei_hbm.at[pl.ds(base, DCH)], cidx)
            pltpu.sync_copy(ones_t, deg_s.at[cidx], add=True)

        plsc.subcore_barrier()

        pltpu.sync_copy(deg_s.at[pl.ds(s * DSTRIPE, DSTRIPE)], stripe_t)
        lane0 = jnp.zeros((16,), jnp.int32)
        rowi = lax.iota(jnp.int32, 16)

        @pl.loop(0, DSTRIPE, step=16)
        def _(r):
            v = plsc.load_gather(stripe_t, [rowi + r, lane0])
            out1d[pl.ds(r, 16)] = v

        pltpu.sync_copy(out1d, deg_hbm.at[c, pl.ds(s * DSTRIPE, DSTRIPE)])

    return deg_kernel(ei_flat)


def _message_sc(g128, ei_flat):
    @pl.kernel(
        out_type=jax.ShapeDtypeStruct((N, 4 * EMB), jnp.float32),
        mesh=_sc_mesh(),
        scratch_types=[
            pltpu.VMEM_SHARED((N, EMB), jnp.float32),
            pltpu.VMEM((2, MCH), jnp.int32),
            pltpu.VMEM((2, MCH), jnp.int32),
            pltpu.VMEM((MCH, EMB), jnp.float32),
            pltpu.VMEM((MCH, EMB), jnp.float32),
            pltpu.SemaphoreType.DMA,
            pltpu.SemaphoreType.DMA,
            pltpu.SemaphoreType.DMA,
            pltpu.SemaphoreType.DMA,
        ],
        compiler_params=_SC_PARAMS_NL,
    )
    def msg_kernel(gview, ei_hbm, a_hbm,
                   acc_s, ib0, ib1, msg0, msg1,
                   is0, is1, gs0, gs1):
        c = lax.axis_index("c")
        s = lax.axis_index("s")

        @pl.loop(0, ZROWS)
        def _(i):
            msg0[i, pl.ds(0, 16)] = jnp.zeros((16,), jnp.float32)
            msg0[i, pl.ds(16, 16)] = jnp.zeros((16,), jnp.float32)

        @pl.loop(0, MSTRIPE // ZROWS)
        def _(k):
            pltpu.sync_copy(
                msg0.at[pl.ds(0, ZROWS)],
                acc_s.at[pl.ds(s * MSTRIPE + k * ZROWS, ZROWS)])

        plsc.subcore_barrier()

        def run():
            bufs = ((ib0, msg0, is0, gs0), (ib1, msg1, is1, gs1))
            base0 = s * MSPANC * MCH

            def idx_descs(j, b):
                ib = bufs[b][0]
                isem = bufs[b][2]
                base = base0 + j * MCH
                return (pltpu.make_async_copy(
                            ei_hbm.at[pl.ds(base, MCH)], ib.at[0], isem),
                        pltpu.make_async_copy(
                            ei_hbm.at[pl.ds(E + base, MCH)], ib.at[1], isem))

            def start_idx(j, b):
                d0, d1 = idx_descs(j, b)
                d0.start()
                d1.start()

            def start_gather(j, b):
                ib, msg, isem, gsem = bufs[b]
                d0, d1 = idx_descs(j, b)
                d0.wait()
                d1.wait()

                @pl.loop(0, MCH, step=16)
                def _(k):
                    v = ib[0, pl.ds(k, 16)]
                    ib[0, pl.ds(k, 16)] = v * 4 + c

                pltpu.async_copy(gview.at[ib.at[0]], msg, gsem)

            def drain(b):
                ib, msg, isem, gsem = bufs[b]
                pltpu.make_async_copy(gview.at[ib.at[0]], msg, gsem).wait()
                pltpu.sync_copy(msg, acc_s.at[ib.at[1]], add=True)

            start_idx(0, 0)
            start_idx(1, 1)

            @pl.loop(0, (MSPANC - 1) // 2)
            def _(i):
                start_gather(2 * i, 0)
                start_gather(2 * i + 1, 1)
                drain(0)
                start_idx(2 * i + 2, 0)
                drain(1)

                @pl.when(2 * i + 3 < MSPANC)
                def _():
                    start_idx(2 * i + 3, 1)

            start_gather(MSPANC - 1, 0)
            drain(0)

            plsc.subcore_barrier()
            stripe = pl.ds(s * MSTRIPE, MSTRIPE)
            pltpu.sync_copy(acc_s.at[stripe],
                            a_hbm.at[stripe, pl.ds(EMB * c, EMB)])

        run()

    return msg_kernel(g128.reshape(4 * N, EMB), ei_flat)


def _fold(gamma, beta, s1, s2):
    m = s1 / N
    v = s2 / N - m * m
    s = gamma * lax.rsqrt(v + EPS)
    return s, beta - m * s


def kernel(high_dim_features, low_dim_features, edge_index,
           bn_low_g, bn_low_b, bn_high_g, bn_high_b,
           W_low, b_low, mlp_low_g, mlp_low_b,
           W_high, b_high, mlp_high_g, mlp_high_b,
           W_gcn, b_gcn, W_cls, b_cls):
    ei_flat = edge_index.reshape(2 * E)
    lowT = low_dim_features.T.reshape(LD, NBLK, 1, BN_ROWS)

    degp = _degree_sc(ei_flat)
    deg3 = (degp[0, :N] + degp[1, :N]).reshape(NBLK, 1, BN_ROWS)

    sh, sl = pl.pallas_call(
        _stats_body,
        grid=(NBLK,),
        in_specs=[
            pl.BlockSpec((BN_ROWS, HD), lambda i: (i, 0)),
            pl.BlockSpec((LD, 1, 1, BN_ROWS), lambda i: (0, i, 0, 0)),
        ],
        out_specs=[
            pl.BlockSpec((8, HD), lambda i: (0, 0)),
            pl.BlockSpec((8, LD), lambda i: (0, 0)),
        ],
        out_shape=[
            jax.ShapeDtypeStruct((8, HD), jnp.float32),
            jax.ShapeDtypeStruct((8, LD), jnp.float32),
        ],
    )(high_dim_features, lowT)

    s_hi, t_hi = _fold(bn_high_g, bn_high_b, sh[0], sh[1])
    s_lo, t_lo = _fold(bn_low_g, bn_low_b, sl[0], sl[1])
    Wh = s_hi[:, None] * W_high
    bh = (t_hi @ W_high + b_high)[None, :]
    Wl = s_lo[:, None] * W_low
    bl = (t_lo @ W_low + b_low)[None, :]

    y, sy = pl.pallas_call(
        _embed_body,
        grid=(NBLK,),
        in_specs=[
            pl.BlockSpec((BN_ROWS, HD), lambda i: (i, 0)),
            pl.BlockSpec((LD, 1, 1, BN_ROWS), lambda i: (0, i, 0, 0)),
            pl.BlockSpec((HD, EMB), lambda i: (0, 0)),
            pl.BlockSpec((1, EMB), lambda i: (0, 0)),
            pl.BlockSpec((LD, EMB), lambda i: (0, 0)),
            pl.BlockSpec((1, EMB), lambda i: (0, 0)),
        ],
        out_specs=[
            pl.BlockSpec((BN_ROWS, HID), lambda i: (i, 0)),
            pl.BlockSpec((8, HID), lambda i: (0, 0)),
        ],
        out_shape=[
            jax.ShapeDtypeStruct((N, HID), jnp.float32),
            jax.ShapeDtypeStruct((8, HID), jnp.float32),
        ],
    )(high_dim_features, lowT, Wh, bh, Wl, bl)

    gy = jnp.concatenate([mlp_high_g, mlp_low_g])
    by = jnp.concatenate([mlp_high_b, mlp_low_b])
    s_y, t_y = _fold(gy, by, sy[0], sy[1])

    g128 = pl.pallas_call(
        _project_body,
        grid=(NBLK,),
        in_specs=[
            pl.BlockSpec((BN_ROWS, HID), lambda i: (i, 0)),
            pl.BlockSpec((1, HID), lambda i: (0, 0)),
            pl.BlockSpec((1, HID), lambda i: (0, 0)),
            pl.BlockSpec((1, 1, BN_ROWS), lambda i: (i, 0, 0)),
        ],
        out_specs=pl.BlockSpec((BN_ROWS, 2 * HID), lambda i: (i, 0)),
        out_shape=jax.ShapeDtypeStruct((N, 2 * HID), jnp.float32),
    )(y, s_y[None, :], t_y[None, :], deg3)

    acc128 = _message_sc(g128, ei_flat)

    out = pl.pallas_call(
        _final_body,
        grid=(NBLK,),
        in_specs=[
            pl.BlockSpec((BN_ROWS, 4 * EMB), lambda i: (i, 0)),
            pl.BlockSpec((BN_ROWS, 2 * HID), lambda i: (i, 0)),
            pl.BlockSpec((1, 1, BN_ROWS), lambda i: (i, 0, 0)),
            pl.BlockSpec((HID, HID), lambda i: (0, 0)),
            pl.BlockSpec((1, HID), lambda i: (0, 0)),
            pl.BlockSpec((HID, OUT), lambda i: (0, 0)),
            pl.BlockSpec((1, OUT), lambda i: (0, 0)),
        ],
        out_specs=pl.BlockSpec((BN_ROWS, OUT), lambda i: (i, 0)),
        out_shape=jax.ShapeDtypeStruct((N, OUT), jnp.float32),
    )(acc128, g128, deg3, W_gcn, b_gcn[None, :], W_cls, b_cls[None, :])

    return out
